# baseline (speedup 1.0000x reference)
import functools

import jax
import jax.numpy as jnp
from jax import lax
from jax.experimental import pallas as pl
from jax.experimental.pallas import tpu as pltpu
from jax.experimental.pallas import tpu_sc as plsc

_H = 384
_W = 384
_HW = _H * _W
_NMAPS = 544
_NW = 32
_MPW = _NMAPS // _NW
_CH = 16384
_NCH = _HW // _CH
_U = 8
_RU = 4


def _sc_argmax(x_hbm, out_hbm, buf0, buf1, res, sem0, sem1):
    ci = lax.axis_index("c")
    si = lax.axis_index("s")
    wid = si * 2 + ci
    ioa = lax.iota(jnp.int32, 16)
    base = wid * (_MPW * _HW)
    bufs = (buf0, buf1)
    sems = (sem0, sem1)
    neg = jnp.full((16,), -jnp.inf, jnp.float32)
    bigi = jnp.full((16,), jnp.int32(1 << 30), jnp.int32)

    def perm(v, idx):
        return lax.gather(
            v, idx[:, None],
            dimension_numbers=lax.GatherDimensionNumbers(
                offset_dims=(), collapsed_slice_dims=(0,), start_index_map=(0,)),
            slice_sizes=(1,), mode=lax.GatherScatterMode.PROMISE_IN_BOUNDS)

    def bmax(v):
        for k in (8, 4, 2, 1):
            v = jnp.maximum(v, perm(v, ioa ^ k))
        return v

    def bmin(v):
        for k in (8, 4, 2, 1):
            v = jnp.minimum(v, perm(v, ioa ^ k))
        return v

    init = (jnp.zeros((16,), jnp.int32), jnp.zeros((16,), jnp.int32))

    @pl.loop(0, _MPW, init_carry=init)
    def map_loop(j, carry):
        resv0, resv1 = carry
        off = base + j * _HW

        h = pltpu.async_copy(x_hbm.at[pl.ds(off, _CH)], buf0, sem0)
        cms = []
        for ch in range(_NCH):
            b = ch % 2
            h.wait()
            if ch + 1 < _NCH:
                nb = (ch + 1) % 2
                h = pltpu.async_copy(
                    x_hbm.at[pl.ds(off + (ch + 1) * _CH, _CH)], bufs[nb], sems[nb])

            def red(i, accs, _b=b):
                return tuple(
                    jnp.maximum(a, bufs[_b][pl.ds((i * _U + k) * 16, 16)])
                    for k, a in enumerate(accs))

            accs = lax.fori_loop(0, _CH // (16 * _U), red,
                                 tuple(neg for _ in range(_U)))
            cvec = accs[0]
            for a in accs[1:]:
                cvec = jnp.maximum(cvec, a)
            cms.append(bmax(cvec)[0])

        m = cms[0]
        for t in range(1, _NCH):
            m = jnp.maximum(m, cms[t])
        mvec = jnp.full((16,), m, jnp.float32)

        wch = jnp.int32(_NCH - 1)
        for ch in range(_NCH - 1, -1, -1):
            wch = jnp.where(cms[ch] == m, jnp.int32(ch), wch)

        pltpu.sync_copy(x_hbm.at[pl.ds(off + wch * _CH, _CH)], buf0)

        def scan(i, best):
            for k in range(_RU):
                p = i * _RU + k
                v = buf0[pl.ds(p * 16, 16)]
                best = jnp.minimum(best, jnp.where(v == mvec, ioa + p * 16, bigi))
            return best

        best = lax.fori_loop(0, _CH // (16 * _RU), scan, bigi)
        bi = bmin(best)[0]
        lin = wch * _CH + bi

        resv0 = jnp.where(ioa == j, lin, resv0)
        resv1 = jnp.where(ioa + 16 == j, lin, resv1)
        return resv0, resv1

    resv0, resv1 = map_loop
    res[pl.ds(0, 16)] = resv0
    res[pl.ds(16, 16)] = resv1
    pltpu.sync_copy(res, out_hbm.at[wid])


def kernel(heatmaps):
    b, p, h, w = heatmaps.shape
    flat = heatmaps.reshape(-1)
    run = functools.partial(
        pl.kernel,
        out_type=jax.ShapeDtypeStruct((_NW, 32), jnp.int32),
        mesh=plsc.VectorSubcoreMesh(core_axis_name="c", subcore_axis_name="s"),
        scratch_types=[
            pltpu.VMEM((_CH,), jnp.float32),
            pltpu.VMEM((_CH,), jnp.float32),
            pltpu.VMEM((32,), jnp.int32),
            pltpu.SemaphoreType.DMA,
            pltpu.SemaphoreType.DMA,
        ],
    )(_sc_argmax)
    lin = run(flat)[:, :_MPW].reshape(b, p)
    wi = lin % w
    hi = lin // w
    return jnp.stack([wi, hi], axis=-1).astype(jnp.int32)


# speedup vs baseline: 1.4570x; 1.4570x over previous
import functools

import jax
import jax.numpy as jnp
from jax import lax
from jax.experimental import pallas as pl
from jax.experimental.pallas import tpu as pltpu
from jax.experimental.pallas import tpu_sc as plsc

_H = 384
_W = 384
_HW = _H * _W
_NW = 32
_K = 96
_MPW = _K // _NW
_CH = 16384
_NCH = _HW // _CH
_U = 8
_RU = 4
_B = 8


def _sc_argmax(x_hbm, out_hbm, buf0, buf1, res, sem0, sem1):
    ci = lax.axis_index("c")
    si = lax.axis_index("s")
    wid = si * 2 + ci
    ioa = lax.iota(jnp.int32, 16)
    base = wid * (_MPW * _HW)
    bufs = (buf0, buf1)
    sems = (sem0, sem1)
    neg = jnp.full((16,), -jnp.inf, jnp.float32)
    bigi = jnp.full((16,), jnp.int32(1 << 30), jnp.int32)

    def perm(v, idx):
        return lax.gather(
            v, idx[:, None],
            dimension_numbers=lax.GatherDimensionNumbers(
                offset_dims=(), collapsed_slice_dims=(0,), start_index_map=(0,)),
            slice_sizes=(1,), mode=lax.GatherScatterMode.PROMISE_IN_BOUNDS)

    def bmax(v):
        for k in (8, 4, 2, 1):
            v = jnp.maximum(v, perm(v, ioa ^ k))
        return v

    def bmin(v):
        for k in (8, 4, 2, 1):
            v = jnp.minimum(v, perm(v, ioa ^ k))
        return v

    init = jnp.zeros((16,), jnp.int32)

    @pl.loop(0, _MPW, init_carry=init)
    def map_loop(j, resv0):
        off = base + j * _HW

        h = pltpu.async_copy(x_hbm.at[pl.ds(off, _CH)], buf0, sem0)
        cms = []
        for ch in range(_NCH):
            b = ch % 2
            h.wait()
            if ch + 1 < _NCH:
                nb = (ch + 1) % 2
                h = pltpu.async_copy(
                    x_hbm.at[pl.ds(off + (ch + 1) * _CH, _CH)], bufs[nb], sems[nb])

            def red(i, accs, _b=b):
                return tuple(
                    jnp.maximum(a, bufs[_b][pl.ds((i * _U + k) * 16, 16)])
                    for k, a in enumerate(accs))

            accs = lax.fori_loop(0, _CH // (16 * _U), red,
                                 tuple(neg for _ in range(_U)))
            cvec = accs[0]
            for a in accs[1:]:
                cvec = jnp.maximum(cvec, a)
            cms.append(bmax(cvec)[0])

        m = cms[0]
        for t in range(1, _NCH):
            m = jnp.maximum(m, cms[t])
        mvec = jnp.full((16,), m, jnp.float32)

        wch = jnp.int32(_NCH - 1)
        for ch in range(_NCH - 1, -1, -1):
            wch = jnp.where(cms[ch] == m, jnp.int32(ch), wch)

        pltpu.sync_copy(x_hbm.at[pl.ds(off + wch * _CH, _CH)], buf0)

        def scan(i, best):
            for k in range(_RU):
                p = i * _RU + k
                v = buf0[pl.ds(p * 16, 16)]
                best = jnp.minimum(best, jnp.where(v == mvec, ioa + p * 16, bigi))
            return best

        best = lax.fori_loop(0, _CH // (16 * _RU), scan, bigi)
        bi = bmin(best)[0]
        lin = wch * _CH + bi

        return jnp.where(ioa == j, lin, resv0)

    res[pl.ds(0, 16)] = map_loop
    pltpu.sync_copy(res, out_hbm.at[wid])


def _sc_part(flat_k):
    run = functools.partial(
        pl.kernel,
        out_type=jax.ShapeDtypeStruct((_NW, 16), jnp.int32),
        mesh=plsc.VectorSubcoreMesh(core_axis_name="c", subcore_axis_name="s"),
        scratch_types=[
            pltpu.VMEM((_CH,), jnp.float32),
            pltpu.VMEM((_CH,), jnp.float32),
            pltpu.VMEM((16,), jnp.int32),
            pltpu.SemaphoreType.DMA,
            pltpu.SemaphoreType.DMA,
        ],
    )(_sc_argmax)
    lin = run(flat_k)[:, :_MPW].reshape(_K)
    wi = lin % _W
    hi = lin // _W
    return jnp.stack([wi, hi], axis=-1).astype(jnp.int32)


def _tc_body(x_ref, o_ref):
    nb, h, w = x_ref.shape
    ch = 32
    r = ch // 8
    nc = h // ch
    big = jnp.int32(1 << 20)

    acc = jnp.max(x_ref[...].reshape(nb, h // 8, 8, w), axis=1)
    m = jnp.max(acc, axis=(1, 2))
    mb = m[:, None, None, None]

    jj = jax.lax.broadcasted_iota(jnp.int32, (1, r, 8, w), 1)
    ss = jax.lax.broadcasted_iota(jnp.int32, (1, r, 8, w), 2)
    rowrel = jj * 8 + ss
    best8 = None
    for i in range(nc):
        c4 = x_ref[:, i * ch:(i + 1) * ch, :].reshape(nb, r, 8, w)
        rel = jnp.min(jnp.where(c4 == mb, rowrel, big), axis=1) + i * ch
        best8 = rel if best8 is None else jnp.minimum(best8, rel)

    col = jax.lax.broadcasted_iota(jnp.int32, (1, 8, w), 2)
    idx = jnp.min(jnp.where(best8 < h, best8 * w + col, big), axis=(1, 2))
    wi = idx % w
    hi = idx // w
    sel = jax.lax.broadcasted_iota(jnp.int32, (1, 1, 2), 2)
    o_ref[...] = jnp.where(sel == 0, wi[:, None, None], hi[:, None, None])


def _tc_part(flat_rest):
    n2 = flat_rest.shape[0]
    out = pl.pallas_call(
        _tc_body,
        grid=(n2 // _B,),
        in_specs=[pl.BlockSpec((_B, _H, _W), lambda i: (i, 0, 0))],
        out_specs=pl.BlockSpec((_B, 1, 2), lambda i: (i, 0, 0)),
        out_shape=jax.ShapeDtypeStruct((n2, 1, 2), jnp.int32),
    )(flat_rest)
    return out.reshape(n2, 2)


def kernel(heatmaps):
    b, p, h, w = heatmaps.shape
    n = b * p
    flat = heatmaps.reshape(n, h, w)
    sc_out = _sc_part(flat[:_K].reshape(-1))
    tc_out = _tc_part(flat[_K:])
    return jnp.concatenate([sc_out, tc_out], axis=0).reshape(b, p, 2)


# speedup vs baseline: 1.4585x; 1.0011x over previous
import functools

import jax
import jax.numpy as jnp
from jax import lax
from jax.experimental import pallas as pl
from jax.experimental.pallas import tpu as pltpu
from jax.experimental.pallas import tpu_sc as plsc

_H = 384
_W = 384
_HW = _H * _W
_NW = 32
_K = 96
_MPW = _K // _NW
_CH = 16384
_NCH = _HW // _CH
_U = 8
_RU = 4
_B = 8


def _sc_argmax(x_hbm, out_hbm, buf0, buf1, res, sem0, sem1):
    ci = lax.axis_index("c")
    si = lax.axis_index("s")
    wid = si * 2 + ci
    ioa = lax.iota(jnp.int32, 16)
    base = wid * (_MPW * _HW)
    bufs = (buf0, buf1)
    sems = (sem0, sem1)
    neg = jnp.full((16,), -jnp.inf, jnp.float32)
    bigi = jnp.full((16,), jnp.int32(1 << 30), jnp.int32)

    def perm(v, idx):
        return lax.gather(
            v, idx[:, None],
            dimension_numbers=lax.GatherDimensionNumbers(
                offset_dims=(), collapsed_slice_dims=(0,), start_index_map=(0,)),
            slice_sizes=(1,), mode=lax.GatherScatterMode.PROMISE_IN_BOUNDS)

    def bmax(v):
        for k in (8, 4, 2, 1):
            v = jnp.maximum(v, perm(v, ioa ^ k))
        return v

    def bmin(v):
        for k in (8, 4, 2, 1):
            v = jnp.minimum(v, perm(v, ioa ^ k))
        return v

    init = jnp.zeros((16,), jnp.int32)

    @pl.loop(0, _MPW, init_carry=init)
    def map_loop(j, resv0):
        off = base + j * _HW

        h = pltpu.async_copy(x_hbm.at[pl.ds(off, _CH)], buf0, sem0)
        cms = []
        for ch in range(_NCH):
            b = ch % 2
            h.wait()
            if ch + 1 < _NCH:
                nb = (ch + 1) % 2
                h = pltpu.async_copy(
                    x_hbm.at[pl.ds(off + (ch + 1) * _CH, _CH)], bufs[nb], sems[nb])

            def red(i, accs, _b=b):
                return tuple(
                    jnp.maximum(a, bufs[_b][pl.ds((i * _U + k) * 16, 16)])
                    for k, a in enumerate(accs))

            accs = lax.fori_loop(0, _CH // (16 * _U), red,
                                 tuple(neg for _ in range(_U)))
            cvec = accs[0]
            for a in accs[1:]:
                cvec = jnp.maximum(cvec, a)
            cms.append(bmax(cvec)[0])

        m = cms[0]
        for t in range(1, _NCH):
            m = jnp.maximum(m, cms[t])
        mvec = jnp.full((16,), m, jnp.float32)

        wch = jnp.int32(_NCH - 1)
        for ch in range(_NCH - 1, -1, -1):
            wch = jnp.where(cms[ch] == m, jnp.int32(ch), wch)

        pltpu.sync_copy(x_hbm.at[pl.ds(off + wch * _CH, _CH)], buf0)

        def scan(i, best):
            for k in range(_RU):
                p = i * _RU + k
                v = buf0[pl.ds(p * 16, 16)]
                best = jnp.minimum(best, jnp.where(v == mvec, ioa + p * 16, bigi))
            return best

        best = lax.fori_loop(0, _CH // (16 * _RU), scan, bigi)
        bi = bmin(best)[0]
        lin = wch * _CH + bi

        return jnp.where(ioa == j, lin, resv0)

    res[pl.ds(0, 16)] = map_loop
    pltpu.sync_copy(res, out_hbm.at[wid])


def _sc_part(flat_k):
    run = functools.partial(
        pl.kernel,
        out_type=jax.ShapeDtypeStruct((_NW, 16), jnp.int32),
        mesh=plsc.VectorSubcoreMesh(core_axis_name="c", subcore_axis_name="s"),
        scratch_types=[
            pltpu.VMEM((_CH,), jnp.float32),
            pltpu.VMEM((_CH,), jnp.float32),
            pltpu.VMEM((16,), jnp.int32),
            pltpu.SemaphoreType.DMA,
            pltpu.SemaphoreType.DMA,
        ],
    )(_sc_argmax)
    lin = run(flat_k)[:, :_MPW].reshape(_K)
    wi = lin % _W
    hi = lin // _W
    return jnp.stack([wi, hi], axis=-1).astype(jnp.int32)


def _tc_body(x_ref, o_ref):
    nb, h, w = x_ref.shape
    ch = 32
    r = ch // 8
    nc = h // ch
    big = jnp.int32(1 << 20)

    acc = jnp.max(x_ref[...].reshape(nb, h // 8, 8, w), axis=1)
    m = jnp.max(acc, axis=(1, 2))
    mb = m[:, None, None, None]

    jj = jax.lax.broadcasted_iota(jnp.int32, (1, r, 8, w), 1)
    ss = jax.lax.broadcasted_iota(jnp.int32, (1, r, 8, w), 2)
    rowrel = jj * 8 + ss
    best8 = None
    for i in range(nc):
        c4 = x_ref[:, i * ch:(i + 1) * ch, :].reshape(nb, r, 8, w)
        rel = jnp.min(jnp.where(c4 == mb, rowrel, big), axis=1) + i * ch
        best8 = rel if best8 is None else jnp.minimum(best8, rel)

    col = jax.lax.broadcasted_iota(jnp.int32, (1, 8, w), 2)
    idx = jnp.min(jnp.where(best8 < h, best8 * w + col, big), axis=(1, 2))
    wi = idx % w
    hi = idx // w
    sel = jax.lax.broadcasted_iota(jnp.int32, (1, 1, 2), 2)
    o_ref[...] = jnp.where(sel == 0, wi[:, None, None], hi[:, None, None])


def _tc_part(flat_rest):
    n2 = flat_rest.shape[0]
    out = pl.pallas_call(
        _tc_body,
        grid=(n2 // _B,),
        in_specs=[pl.BlockSpec((_B, _H, _W), lambda i: (i, 0, 0))],
        out_specs=pl.BlockSpec((_B, 1, 2), lambda i: (i, 0, 0)),
        out_shape=jax.ShapeDtypeStruct((n2, 1, 2), jnp.int32),
    )(flat_rest)
    return out.reshape(n2, 2)


def kernel(heatmaps):
    b, p, h, w = heatmaps.shape
    n = b * p
    flat = heatmaps.reshape(n, h, w)
    tc_out = _tc_part(flat[_K:])
    sc_out = _sc_part(flat[:_K].reshape(-1))
    return jnp.concatenate([sc_out, tc_out], axis=0).reshape(b, p, 2)


# speedup vs baseline: 5.5224x; 3.7864x over previous
import jax
import jax.numpy as jnp
from jax.experimental import pallas as pl

_B = 16


def _argmax_body(x_ref, o_ref):
    nb, h, w = x_ref.shape
    ch = 32
    r = ch // 8
    nc = h // ch
    big = jnp.int32(1 << 20)

    acc = jnp.max(x_ref[...].reshape(nb, h // 8, 8, w), axis=1)
    m = jnp.max(acc, axis=(1, 2))
    mb = m[:, None, None, None]

    jj = jax.lax.broadcasted_iota(jnp.int32, (1, r, 8, w), 1)
    ss = jax.lax.broadcasted_iota(jnp.int32, (1, r, 8, w), 2)
    rowrel = jj * 8 + ss
    best8 = None
    for i in range(nc):
        c4 = x_ref[:, i * ch:(i + 1) * ch, :].reshape(nb, r, 8, w)
        rel = jnp.min(jnp.where(c4 == mb, rowrel, big), axis=1) + i * ch
        best8 = rel if best8 is None else jnp.minimum(best8, rel)

    col = jax.lax.broadcasted_iota(jnp.int32, (1, 8, w), 2)
    idx = jnp.min(jnp.where(best8 < h, best8 * w + col, big), axis=(1, 2))
    wi = idx % w
    hi = idx // w
    sel = jax.lax.broadcasted_iota(jnp.int32, (1, 1, 2), 2)
    o_ref[...] = jnp.where(sel == 0, wi[:, None, None], hi[:, None, None])


def kernel(heatmaps):
    b, p, h, w = heatmaps.shape
    n = b * p
    flat = heatmaps.reshape(n, h, w)
    out = pl.pallas_call(
        _argmax_body,
        grid=(n // _B,),
        in_specs=[pl.BlockSpec((_B, h, w), lambda i: (i, 0, 0))],
        out_specs=pl.BlockSpec((_B, 1, 2), lambda i: (i, 0, 0)),
        out_shape=jax.ShapeDtypeStruct((n, 1, 2), jnp.int32),
    )(flat)
    return out.reshape(b, p, 2)


# speedup vs baseline: 5.7934x; 1.0491x over previous
import jax
import jax.numpy as jnp
from jax.experimental import pallas as pl

_B = 32


def _argmax_body(x_ref, o_ref):
    nb, h, w = x_ref.shape
    ch = 32
    r = ch // 8
    nc = h // ch
    big = jnp.int32(1 << 20)

    acc = jnp.max(x_ref[...].reshape(nb, h // 8, 8, w), axis=1)
    m = jnp.max(acc, axis=(1, 2))
    mb = m[:, None, None, None]

    jj = jax.lax.broadcasted_iota(jnp.int32, (1, r, 8, w), 1)
    ss = jax.lax.broadcasted_iota(jnp.int32, (1, r, 8, w), 2)
    rowrel = jj * 8 + ss
    best8 = None
    for i in range(nc):
        c4 = x_ref[:, i * ch:(i + 1) * ch, :].reshape(nb, r, 8, w)
        rel = jnp.min(jnp.where(c4 == mb, rowrel, big), axis=1) + i * ch
        best8 = rel if best8 is None else jnp.minimum(best8, rel)

    col = jax.lax.broadcasted_iota(jnp.int32, (1, 8, w), 2)
    idx = jnp.min(jnp.where(best8 < h, best8 * w + col, big), axis=(1, 2))
    wi = idx % w
    hi = idx // w
    sel = jax.lax.broadcasted_iota(jnp.int32, (1, 1, 2), 2)
    o_ref[...] = jnp.where(sel == 0, wi[:, None, None], hi[:, None, None])


def kernel(heatmaps):
    b, p, h, w = heatmaps.shape
    n = b * p
    flat = heatmaps.reshape(n, h, w)
    out = pl.pallas_call(
        _argmax_body,
        grid=(n // _B,),
        in_specs=[pl.BlockSpec((_B, h, w), lambda i: (i, 0, 0))],
        out_specs=pl.BlockSpec((_B, 1, 2), lambda i: (i, 0, 0)),
        out_shape=jax.ShapeDtypeStruct((n, 1, 2), jnp.int32),
    )(flat)
    return out.reshape(b, p, 2)
